# Initial kernel scaffold; baseline (speedup 1.0000x reference)
import jax
import jax.numpy as jnp
from jax.experimental import pallas as pl


def kernel(input_ids, word_table, pos_table, ln_gamma, ln_beta, W, b):
    raise NotImplementedError("write your pallas kernel here")


# speedup vs baseline: 1.2585x; 1.2585x over previous
import functools

import jax
import jax.numpy as jnp
from jax import lax
from jax.experimental import pallas as pl
from jax.experimental.pallas import tpu as pltpu
from jax.experimental.pallas import tpu_sc as plsc

_NUM_SC = 2
_NUM_SUBCORES = 16
_NUM_WORKERS = _NUM_SC * _NUM_SUBCORES


def _make_sc_gather(n_tokens: int, embed: int, chunk: int):
    per_worker = n_tokens // _NUM_WORKERS
    assert per_worker % chunk == 0
    n_chunks = per_worker // chunk
    mesh = plsc.VectorSubcoreMesh(core_axis_name="c", subcore_axis_name="s")

    @functools.partial(
        pl.kernel,
        mesh=mesh,
        out_type=jax.ShapeDtypeStruct((n_tokens, embed), jnp.float32),
        scratch_types=[
            pltpu.VMEM((chunk,), jnp.int32),
            pltpu.VMEM((chunk, embed), jnp.float32),
            pltpu.SemaphoreType.DMA,
        ],
    )
    def gather(table_hbm, idx_hbm, out_hbm, idx_v, rows_v, sem):
        wid = lax.axis_index("s") * _NUM_SC + lax.axis_index("c")
        for c in range(n_chunks):
            base = wid * per_worker + c * chunk
            pltpu.sync_copy(idx_hbm.at[pl.ds(base, chunk)], idx_v)
            pltpu.async_copy(table_hbm.at[idx_v], rows_v, sem).wait()
            pltpu.sync_copy(rows_v, out_hbm.at[pl.ds(base, chunk)])

    return gather


def _ln_matmul_body(x_ref, pos_ref, g_ref, bt_ref, w_ref, bias_ref, o_ref):
    x = x_ref[...] + pos_ref[...]
    mu = jnp.mean(x, axis=-1, keepdims=True)
    xc = x - mu
    var = jnp.mean(xc * xc, axis=-1, keepdims=True)
    xn = xc * lax.rsqrt(var + 1e-12)
    xn = xn * g_ref[...] + bt_ref[...]
    o_ref[...] = (
        jnp.dot(xn, w_ref[...], preferred_element_type=jnp.float32)
        + bias_ref[...]
    )


def _make_tc_fused(n_tokens: int, seq: int, embed: int, hidden: int, tm: int):
    grid = (n_tokens // tm,)
    pos_blocks = seq // tm

    return pl.pallas_call(
        _ln_matmul_body,
        grid=grid,
        in_specs=[
            pl.BlockSpec((tm, embed), lambda i: (i, 0)),
            pl.BlockSpec((tm, embed), lambda i: (i % pos_blocks, 0)),
            pl.BlockSpec((1, embed), lambda i: (0, 0)),
            pl.BlockSpec((1, embed), lambda i: (0, 0)),
            pl.BlockSpec((embed, hidden), lambda i: (0, 0)),
            pl.BlockSpec((1, hidden), lambda i: (0, 0)),
        ],
        out_specs=pl.BlockSpec((tm, hidden), lambda i: (i, 0)),
        out_shape=jax.ShapeDtypeStruct((n_tokens, hidden), jnp.float32),
    )


def kernel(input_ids, word_table, pos_table, ln_gamma, ln_beta, W, b):
    bsz, seq = input_ids.shape
    vocab, embed = word_table.shape
    hidden = W.shape[1]
    n_tokens = bsz * seq

    ids_flat = input_ids.reshape(n_tokens).astype(jnp.int32)
    gathered = _make_sc_gather(n_tokens, embed, chunk=64)(word_table, ids_flat)

    fused = _make_tc_fused(n_tokens, seq, embed, hidden, tm=256)
    out = fused(
        gathered,
        pos_table[:seq],
        ln_gamma.reshape(1, embed),
        ln_beta.reshape(1, embed),
        W,
        b.reshape(1, hidden),
    )
    return out.reshape(bsz, seq, hidden)
